# baseline (speedup 1.0000x reference)
import functools

import jax
import jax.numpy as jnp
from jax import lax
from jax.experimental import pallas as pl
from jax.experimental.pallas import tpu as pltpu
from jax.experimental.pallas import tpu_sc as plsc

D_MODEL = 128
H = 30
W = 30
NCOLORS = 11
P = H * W
B = 1024
NPL = B * H
NC, NS = 2, 16
NW = NC * NS
PPW = NPL // NW
BPW = B // NW
PLCH = 4
NCH = PPW // PLCH
CPW = PPW * W
NBUF = 4


def _fused_body(color_ref, row_ref, col_ref, out_ref):
    out_ref[...] = (color_ref[...][:, None, None, :]
                    + row_ref[...][None, :, None, :]
                    + col_ref[...][None, None, :, :])


def _build_fused(color_table, row_table, col_table):
    out = pl.pallas_call(
        _fused_body,
        out_shape=jax.ShapeDtypeStruct((NCOLORS, H, W, D_MODEL), jnp.float32),
    )(color_table, row_table, col_table)
    return out.reshape(NCOLORS * P, D_MODEL)


_mesh = plsc.VectorSubcoreMesh(core_axis_name="c", subcore_axis_name="s",
                               num_cores=NC, num_subcores=NS)


@functools.partial(
    pl.kernel,
    out_type=jax.ShapeDtypeStruct((B, H, W, D_MODEL), jnp.float32),
    mesh=_mesh,
    compiler_params=pltpu.CompilerParams(use_tc_tiling_on_sc=True),
    scratch_types=[
        pltpu.VMEM((CPW + 16,), jnp.int32),
        pltpu.VMEM((NCH, PLCH * 32), jnp.int32),
        pltpu.VMEM((NBUF, PLCH, W, D_MODEL), jnp.float32),
        [pltpu.SemaphoreType.DMA] * NBUF,
        [pltpu.SemaphoreType.DMA] * NBUF,
    ],
)
def _sc_gather(fused_hbm, grid_hbm, out_hbm, grid_v, idx_v, rows_v,
               gsems, ssems):
    wid = lax.axis_index("s") * NC + lax.axis_index("c")
    bbase = wid * BPW

    pltpu.sync_copy(grid_hbm.at[pl.ds(wid * CPW, CPW)],
                    grid_v.at[pl.ds(0, CPW)])

    iota = lax.iota(jnp.int32, 16)

    def idx_body(c, h0):
        for k in range(PLCH):
            hk = h0 + k
            hk = jnp.where(hk >= H, hk - H, hk)
            f = c * (PLCH * W) + k * W
            pb = hk * W + iota
            idx_v[c, pl.ds(k * 32, 16)] = grid_v[pl.ds(f, 16)] * P + pb
            idx_v[c, pl.ds(k * 32 + 16, 16)] = (
                grid_v[pl.ds(f + 16, 16)] * P + pb + 16)
        h1 = h0 + PLCH
        return jnp.where(h1 >= H, h1 - H, h1)

    lax.fori_loop(0, NCH, idx_body, jnp.int32(0))

    def g_descs(c, b):
        return [pltpu.make_async_copy(
                    fused_hbm.at[idx_v.at[c, pl.ds(k * 32, W)]],
                    rows_v.at[b, k], gsems[b])
                for k in range(PLCH)]

    def start_gather(c, b):
        for d in g_descs(c, b):
            d.start()

    def wait_gather(c, b):
        for d in g_descs(c, b):
            d.wait()

    def s_start(b, bloc, h0):
        bg = bbase + bloc

        @pl.when(h0 != H - 2)
        def _():
            pltpu.make_async_copy(
                rows_v.at[b], out_hbm.at[bg, pl.ds(h0, PLCH)],
                ssems[b]).start()

        @pl.when(h0 == H - 2)
        def _():
            pltpu.make_async_copy(
                rows_v.at[b, pl.ds(0, 2)],
                out_hbm.at[bg, pl.ds(H - 2, 2)], ssems[b]).start()
            pltpu.make_async_copy(
                rows_v.at[b, pl.ds(2, 2)],
                out_hbm.at[bg + 1, pl.ds(0, 2)], ssems[b]).start()

    def s_start_static(c, b):
        h0 = (c * PLCH) % H
        assert h0 != H - 2
        pltpu.make_async_copy(
            rows_v.at[b],
            out_hbm.at[bbase + (c * PLCH) // H, pl.ds(h0, PLCH)],
            ssems[b]).start()

    def s_wait(b):
        pltpu.make_async_copy(
            rows_v.at[b], out_hbm.at[0, pl.ds(0, PLCH)], ssems[b]).wait()

    start_gather(0, 0)
    for c in range(NBUF):
        b = c % NBUF
        wait_gather(c, b)
        s_start_static(c, b)
        nb = (b + 1) % NBUF
        if c == NBUF - 1:
            s_wait(nb)
        start_gather(c + 1, nb)

    def outer(t, state):
        bloc, h0 = state
        for b in range(NBUF):
            c = t * NBUF + b
            wait_gather(c, b)
            s_start(b, bloc, h0)
            nb = (b + 1) % NBUF
            s_wait(nb)
            start_gather(c + 1, nb)
            h1 = h0 + PLCH
            wrap = h1 >= H
            h0 = jnp.where(wrap, h1 - H, h1)
            bloc = bloc + wrap.astype(jnp.int32)
        return bloc, h0

    c0 = NBUF
    lax.fori_loop(1, NCH // NBUF - 1, outer,
                  (jnp.int32((c0 * PLCH) // H), jnp.int32((c0 * PLCH) % H)))

    for c in range(NCH - NBUF, NCH):
        b = c % NBUF
        wait_gather(c, b)
        s_start_static(c, b)
        if c + 1 < NCH:
            nb = (b + 1) % NBUF
            s_wait(nb)
            start_gather(c + 1, nb)
    for c in range(NCH - NBUF, NCH):
        s_wait(c % NBUF)


def kernel(grid, color_table, row_table, col_table):
    fused = _build_fused(color_table, row_table, col_table)
    return _sc_gather(fused, grid.reshape(B * P))


# speedup vs baseline: 1.0002x; 1.0002x over previous
import functools

import jax
import jax.numpy as jnp
from jax import lax
from jax.experimental import pallas as pl
from jax.experimental.pallas import tpu as pltpu
from jax.experimental.pallas import tpu_sc as plsc

D_MODEL = 128
H = 30
W = 30
NCOLORS = 11
P = H * W
B = 1024
NPL = B * H
NC, NS = 2, 16
NW = NC * NS
PPW = NPL // NW
BPW = B // NW
PLCH = 4
NCH = PPW // PLCH
CPW = PPW * W
NBUF = 3


def _fused_body(color_ref, row_ref, col_ref, out_ref):
    out_ref[...] = (color_ref[...][:, None, None, :]
                    + row_ref[...][None, :, None, :]
                    + col_ref[...][None, None, :, :])


def _build_fused(color_table, row_table, col_table):
    out = pl.pallas_call(
        _fused_body,
        out_shape=jax.ShapeDtypeStruct((NCOLORS, H, W, D_MODEL), jnp.float32),
    )(color_table, row_table, col_table)
    return out.reshape(NCOLORS * P, D_MODEL)


_mesh = plsc.VectorSubcoreMesh(core_axis_name="c", subcore_axis_name="s",
                               num_cores=NC, num_subcores=NS)


@functools.partial(
    pl.kernel,
    out_type=jax.ShapeDtypeStruct((B, H, W, D_MODEL), jnp.float32),
    mesh=_mesh,
    compiler_params=pltpu.CompilerParams(use_tc_tiling_on_sc=True),
    scratch_types=[
        pltpu.VMEM((CPW + 16,), jnp.int32),
        pltpu.VMEM((NCH, PLCH * 32), jnp.int32),
        pltpu.VMEM((NBUF, PLCH, W, D_MODEL), jnp.float32),
        [pltpu.SemaphoreType.DMA] * NBUF,
        [pltpu.SemaphoreType.DMA] * NBUF,
    ],
)
def _sc_gather(fused_hbm, grid_hbm, out_hbm, grid_v, idx_v, rows_v,
               gsems, ssems):
    wid = lax.axis_index("s") * NC + lax.axis_index("c")
    bbase = wid * BPW

    pltpu.sync_copy(grid_hbm.at[pl.ds(wid * CPW, CPW)],
                    grid_v.at[pl.ds(0, CPW)])

    iota = lax.iota(jnp.int32, 16)

    def idx_body(c, h0):
        for k in range(PLCH):
            hk = h0 + k
            hk = jnp.where(hk >= H, hk - H, hk)
            f = c * (PLCH * W) + k * W
            pb = hk * W + iota
            idx_v[c, pl.ds(k * 32, 16)] = grid_v[pl.ds(f, 16)] * P + pb
            idx_v[c, pl.ds(k * 32 + 16, 16)] = (
                grid_v[pl.ds(f + 16, 16)] * P + pb + 16)
        h1 = h0 + PLCH
        return jnp.where(h1 >= H, h1 - H, h1)

    lax.fori_loop(0, NCH, idx_body, jnp.int32(0))

    def g_descs(c, b):
        return [pltpu.make_async_copy(
                    fused_hbm.at[idx_v.at[c, pl.ds(k * 32, W)]],
                    rows_v.at[b, k], gsems[b])
                for k in range(PLCH)]

    def start_gather(c, b):
        for d in g_descs(c, b):
            d.start()

    def wait_gather(c, b):
        for d in g_descs(c, b):
            d.wait()

    def s_start(b, bloc, h0):
        bg = bbase + bloc

        @pl.when(h0 != H - 2)
        def _():
            pltpu.make_async_copy(
                rows_v.at[b], out_hbm.at[bg, pl.ds(h0, PLCH)],
                ssems[b]).start()

        @pl.when(h0 == H - 2)
        def _():
            pltpu.make_async_copy(
                rows_v.at[b, pl.ds(0, 2)],
                out_hbm.at[bg, pl.ds(H - 2, 2)], ssems[b]).start()
            pltpu.make_async_copy(
                rows_v.at[b, pl.ds(2, 2)],
                out_hbm.at[bg + 1, pl.ds(0, 2)], ssems[b]).start()

    def s_start_static(c, b):
        h0 = (c * PLCH) % H
        assert h0 != H - 2
        pltpu.make_async_copy(
            rows_v.at[b],
            out_hbm.at[bbase + (c * PLCH) // H, pl.ds(h0, PLCH)],
            ssems[b]).start()

    def s_wait(b):
        pltpu.make_async_copy(
            rows_v.at[b], out_hbm.at[0, pl.ds(0, PLCH)], ssems[b]).wait()

    start_gather(0, 0)
    for c in range(NBUF):
        b = c % NBUF
        wait_gather(c, b)
        s_start_static(c, b)
        nb = (b + 1) % NBUF
        if c == NBUF - 1:
            s_wait(nb)
        start_gather(c + 1, nb)

    def outer(t, state):
        bloc, h0 = state
        for b in range(NBUF):
            c = t * NBUF + b
            wait_gather(c, b)
            s_start(b, bloc, h0)
            nb = (b + 1) % NBUF
            s_wait(nb)
            start_gather(c + 1, nb)
            h1 = h0 + PLCH
            wrap = h1 >= H
            h0 = jnp.where(wrap, h1 - H, h1)
            bloc = bloc + wrap.astype(jnp.int32)
        return bloc, h0

    c0 = NBUF
    lax.fori_loop(1, NCH // NBUF - 1, outer,
                  (jnp.int32((c0 * PLCH) // H), jnp.int32((c0 * PLCH) % H)))

    for c in range(NCH - NBUF, NCH):
        b = c % NBUF
        wait_gather(c, b)
        s_start_static(c, b)
        if c + 1 < NCH:
            nb = (b + 1) % NBUF
            s_wait(nb)
            start_gather(c + 1, nb)
    for c in range(NCH - NBUF, NCH):
        s_wait(c % NBUF)


def kernel(grid, color_table, row_table, col_table):
    fused = _build_fused(color_table, row_table, col_table)
    return _sc_gather(fused, grid.reshape(B * P))
